# baseline (speedup 1.0000x reference)
import jax
import jax.numpy as jnp
from jax.experimental import pallas as pl
from jax.experimental.pallas import tpu as pltpu
from jax.experimental.pallas import tpu_sc as plsc

_LANES = 16


def kernel(x):
    n, r, c = x.shape
    BR = 16
    jb = r // BR
    x2 = x.reshape(n * r, c)

    mesh = plsc.VectorSubcoreMesh(core_axis_name="core", subcore_axis_name="subcore")

    @pl.kernel(out_type=jax.ShapeDtypeStruct((n * r, c), x.dtype), mesh=mesh)
    def sc_reverse(x_hbm, o_hbm):
        def body(in_vmem, out_vmem):
            @pl.loop(0, BR)
            def _(c0):
                @pl.loop(0, c, step=_LANES)
                def _(c1):
                    slc = (pl.ds(c0, 1), pl.ds(c1, _LANES))
                    out_vmem.at[*slc][...] = in_vmem.at[*slc][...]

        pltpu.emit_pipeline(
            body,
            grid=(n, jb),
            in_specs=[pl.BlockSpec((BR, c), lambda i, j: ((n - 1 - i) * jb + j, 0))],
            out_specs=[pl.BlockSpec((BR, c), lambda i, j: (i * jb + j, 0))],
            core_axis_name=("core", "subcore"),
            dimension_semantics=(pltpu.PARALLEL, pltpu.PARALLEL),
        )(x_hbm, o_hbm)

    return sc_reverse(x2).reshape(n, r, c)


# speedup vs baseline: 1.2730x; 1.2730x over previous
import jax
import jax.numpy as jnp
from jax.experimental import pallas as pl
from jax.experimental.pallas import tpu as pltpu
from jax.experimental.pallas import tpu_sc as plsc

_LANES = 16


def kernel(x):
    n, r, c = x.shape
    BR = 16
    jb = r // BR
    x2 = x.reshape(n * r, c)

    mesh = plsc.VectorSubcoreMesh(core_axis_name="core", subcore_axis_name="subcore")

    @pl.kernel(out_type=jax.ShapeDtypeStruct((n * r, c), x.dtype), mesh=mesh)
    def sc_reverse(x_hbm, o_hbm):
        def body(in_vmem, out_vmem):
            @pl.loop(0, BR)
            def _(c0):
                @pl.loop(0, c, step=_LANES, unroll=True)
                def _(c1):
                    slc = (pl.ds(c0, 1), pl.ds(c1, _LANES))
                    out_vmem.at[*slc][...] = in_vmem.at[*slc][...]

        pltpu.emit_pipeline(
            body,
            grid=(n, jb),
            in_specs=[pl.BlockSpec((BR, c), lambda i, j: ((n - 1 - i) * jb + j, 0))],
            out_specs=[pl.BlockSpec((BR, c), lambda i, j: (i * jb + j, 0))],
            core_axis_name=("core", "subcore"),
            dimension_semantics=(pltpu.PARALLEL, pltpu.PARALLEL),
        )(x_hbm, o_hbm)

    return sc_reverse(x2).reshape(n, r, c)
